# baseline (speedup 1.0000x reference)
import functools

import jax
import jax.numpy as jnp
from jax import lax
from jax.experimental import pallas as pl
from jax.experimental.pallas import tpu as pltpu
from jax.experimental.pallas import tpu_sc as plsc

_NC = 2
_NS = 16
_NW = _NC * _NS
_LANES = 16


def _sc_add(B, L, D):
    rows_per_w = L // _NW
    sub = 16
    n_sub = rows_per_w // sub
    mesh = plsc.VectorSubcoreMesh(core_axis_name="c", subcore_axis_name="s")

    @functools.partial(
        pl.kernel,
        mesh=mesh,
        out_type=jax.ShapeDtypeStruct((B * L * D,), jnp.float32),
        scratch_types=[
            pltpu.VMEM((rows_per_w * D,), jnp.float32),
            pltpu.VMEM((sub * D,), jnp.float32),
            pltpu.SemaphoreType.DMA,
        ],
    )
    def k(x_hbm, t_hbm, o_hbm, tch, xb, sem):
        wid = lax.axis_index("s") * _NC + lax.axis_index("c")
        l0 = wid * rows_per_w
        pltpu.sync_copy(t_hbm.at[pl.ds(l0 * D, rows_per_w * D)], tch)

        def batch_body(b, carry):
            def sub_body(s, carry2):
                base = (b * L + l0 + s * sub) * D
                pltpu.sync_copy(x_hbm.at[pl.ds(base, sub * D)], xb)
                toff = s * sub * D

                def add_body(j, carry3):
                    off = j * _LANES
                    xb[pl.ds(off, _LANES)] = (
                        xb[pl.ds(off, _LANES)] + tch[pl.ds(toff + off, _LANES)]
                    )
                    return carry3

                lax.fori_loop(0, sub * D // _LANES, add_body, 0, unroll=8)
                pltpu.sync_copy(xb, o_hbm.at[pl.ds(base, sub * D)])
                return carry2

            return lax.fori_loop(0, n_sub, sub_body, carry)

        lax.fori_loop(0, B, batch_body, 0)

    return k


def kernel(x, table):
    B, L, D = x.shape
    sc = _sc_add(B, L, D)
    out = sc(x.reshape(B * L * D), table.reshape(L * D))
    return out.reshape(B, L, D)


# speedup vs baseline: 1.6323x; 1.6323x over previous
import functools

import jax
import jax.numpy as jnp
from jax import lax
from jax.experimental import pallas as pl
from jax.experimental.pallas import tpu as pltpu
from jax.experimental.pallas import tpu_sc as plsc

_NC = 2
_NS = 16
_NW = _NC * _NS
_LANES = 16
_NBUF = 3


def _sc_add(B, L, D):
    rows_per_w = L // _NW
    sub = 16
    n_steps = B * (rows_per_w // sub)
    mesh = plsc.VectorSubcoreMesh(core_axis_name="c", subcore_axis_name="s")

    @functools.partial(
        pl.kernel,
        mesh=mesh,
        out_type=jax.ShapeDtypeStruct((B * L * D,), jnp.float32),
        scratch_types=[
            pltpu.VMEM((rows_per_w * D,), jnp.float32),
            [pltpu.VMEM((sub * D,), jnp.float32) for _ in range(_NBUF)],
            pltpu.SemaphoreType.DMA,
            [pltpu.SemaphoreType.DMA for _ in range(_NBUF)],
            [pltpu.SemaphoreType.DMA for _ in range(_NBUF)],
        ],
    )
    def k(x_hbm, t_hbm, o_hbm, tch, bufs, sem_t, sems_in, sems_out):
        wid = lax.axis_index("s") * _NC + lax.axis_index("c")
        l0 = wid * rows_per_w
        n_sub = rows_per_w // sub

        def hbm_base(i):
            b, s = divmod(i, n_sub)
            return (b * L + l0 + s * sub) * D

        t_dma = pltpu.async_copy(
            t_hbm.at[pl.ds(l0 * D, rows_per_w * D)], tch, sem_t
        )

        pend_in = {}
        pend_out = {}
        for j in range(_NBUF):
            pend_in[j] = pltpu.async_copy(
                x_hbm.at[pl.ds(hbm_base(j), sub * D)], bufs[j], sems_in[j]
            )
        t_dma.wait()

        for i in range(n_steps):
            nxt = i + 1
            if nxt < n_steps and nxt >= _NBUF:
                pend_out[nxt - _NBUF].wait()
                pend_in[nxt] = pltpu.async_copy(
                    x_hbm.at[pl.ds(hbm_base(nxt), sub * D)],
                    bufs[nxt % _NBUF],
                    sems_in[nxt % _NBUF],
                )
            pend_in[i].wait()
            xb = bufs[i % _NBUF]
            toff = (i % n_sub) * sub * D

            def add_body(j, carry, xb=xb, toff=toff):
                off = j * _LANES
                plsc.addupdate(
                    xb.at[pl.ds(off, _LANES)], tch[pl.ds(toff + off, _LANES)]
                )
                return carry

            lax.fori_loop(0, sub * D // _LANES, add_body, 0, unroll=16)
            pend_out[i] = pltpu.async_copy(
                xb, o_hbm.at[pl.ds(hbm_base(i), sub * D)], sems_out[i % _NBUF]
            )

        for i in range(n_steps - _NBUF, n_steps):
            pend_out[i].wait()

    return k


def kernel(x, table):
    B, L, D = x.shape
    sc = _sc_add(B, L, D)
    out = sc(x.reshape(B * L * D), table.reshape(L * D))
    return out.reshape(B, L, D)


# speedup vs baseline: 2.3945x; 1.4669x over previous
import functools

import jax
import jax.numpy as jnp
from jax import lax
from jax.experimental import pallas as pl
from jax.experimental.pallas import tpu as pltpu
from jax.experimental.pallas import tpu_sc as plsc

_NC = 2
_NS = 16
_NW = _NC * _NS
_LANES = 16
_NBUF = 3


def _sc_add(B, L, D):
    rows_per_w = L // _NW
    sub = 16
    n_sub = rows_per_w // sub
    n_steps = B * n_sub
    mesh = plsc.VectorSubcoreMesh(core_axis_name="c", subcore_axis_name="s")

    @functools.partial(
        pl.kernel,
        mesh=mesh,
        out_type=jax.ShapeDtypeStruct((B * L, D), jnp.float32),
        scratch_types=[
            pltpu.VMEM((rows_per_w, D), jnp.float32),
            [pltpu.VMEM((sub, D), jnp.float32) for _ in range(_NBUF)],
            pltpu.SemaphoreType.DMA,
            [pltpu.SemaphoreType.DMA for _ in range(_NBUF)],
            [pltpu.SemaphoreType.DMA for _ in range(_NBUF)],
        ],
    )
    def k(x_hbm, t_hbm, o_hbm, tch, bufs, sem_t, sems_in, sems_out):
        wid = lax.axis_index("s") * _NC + lax.axis_index("c")
        l0 = wid * rows_per_w

        def hbm_row(i):
            b, s = divmod(i, n_sub)
            return b * L + l0 + s * sub

        t_dma = pltpu.async_copy(t_hbm.at[pl.ds(l0, rows_per_w)], tch, sem_t)

        pend_in = {}
        pend_out = {}
        for j in range(_NBUF):
            pend_in[j] = pltpu.async_copy(
                x_hbm.at[pl.ds(hbm_row(j), sub)], bufs[j], sems_in[j]
            )
        t_dma.wait()

        for i in range(n_steps):
            nxt = i + 1
            if nxt < n_steps and nxt >= _NBUF:
                pend_out[nxt - _NBUF].wait()
                pend_in[nxt] = pltpu.async_copy(
                    x_hbm.at[pl.ds(hbm_row(nxt), sub)],
                    bufs[nxt % _NBUF],
                    sems_in[nxt % _NBUF],
                )
            pend_in[i].wait()
            xb = bufs[i % _NBUF]
            trow0 = (i % n_sub) * sub

            def add_body(j, carry, xb=xb, trow0=trow0):
                r = j // (D // _LANES)
                c = (j % (D // _LANES)) * _LANES
                plsc.addupdate(
                    xb.at[r, pl.ds(c, _LANES)],
                    tch[trow0 + r, pl.ds(c, _LANES)],
                )
                return carry

            lax.fori_loop(0, sub * D // _LANES, add_body, 0, unroll=16)
            pend_out[i] = pltpu.async_copy(
                xb, o_hbm.at[pl.ds(hbm_row(i), sub)], sems_out[i % _NBUF]
            )

        for i in range(n_steps - _NBUF, n_steps):
            pend_out[i].wait()

    return k


def kernel(x, table):
    B, L, D = x.shape
    sc = _sc_add(B, L, D)
    out = sc(x.reshape(B * L, D), table)
    return out.reshape(B, L, D)


# speedup vs baseline: 4.0065x; 1.6732x over previous
import functools

import jax
import jax.numpy as jnp
from jax import lax
from jax.experimental import pallas as pl
from jax.experimental.pallas import tpu as pltpu
from jax.experimental.pallas import tpu_sc as plsc

_NC = 2
_NS = 16
_NW = _NC * _NS
_LANES = 16
_NBUF = 3


def _sc_add(B, L, D):
    rows_per_w = L // _NW
    sub = 16
    n_sub = rows_per_w // sub
    n_steps = B * n_sub
    mesh = plsc.VectorSubcoreMesh(core_axis_name="c", subcore_axis_name="s")

    @functools.partial(
        pl.kernel,
        mesh=mesh,
        out_type=jax.ShapeDtypeStruct((B * L, D), jnp.float32),
        scratch_types=[
            pltpu.VMEM((rows_per_w, D), jnp.float32),
            [pltpu.VMEM((sub, D), jnp.float32) for _ in range(_NBUF)],
            pltpu.SemaphoreType.DMA,
            [pltpu.SemaphoreType.DMA for _ in range(_NBUF)],
            [pltpu.SemaphoreType.DMA for _ in range(_NBUF)],
        ],
    )
    def k(x_hbm, t_hbm, o_hbm, tch, bufs, sem_t, sems_in, sems_out):
        wid = lax.axis_index("s") * _NC + lax.axis_index("c")
        l0 = wid * rows_per_w

        def hbm_row(i):
            b, s = divmod(i, n_sub)
            return b * L + l0 + s * sub

        t_dma = pltpu.async_copy(t_hbm.at[pl.ds(l0, rows_per_w)], tch, sem_t)

        pend_in = {}
        pend_out = {}
        for j in range(_NBUF):
            pend_in[j] = pltpu.async_copy(
                x_hbm.at[pl.ds(hbm_row(j), sub)], bufs[j], sems_in[j]
            )
        t_dma.wait()

        for i in range(n_steps):
            nxt = i + 1
            if nxt < n_steps and nxt >= _NBUF:
                pend_out[nxt - _NBUF].wait()
                pend_in[nxt] = pltpu.async_copy(
                    x_hbm.at[pl.ds(hbm_row(nxt), sub)],
                    bufs[nxt % _NBUF],
                    sems_in[nxt % _NBUF],
                )
            pend_in[i].wait()
            xb = bufs[i % _NBUF]
            trow0 = (i % n_sub) * sub

            def add_body(j, xb=xb, trow0=trow0):
                r = j // (D // _LANES)
                c = (j % (D // _LANES)) * _LANES
                plsc.addupdate(
                    xb.at[r, pl.ds(c, _LANES)],
                    tch[trow0 + r, pl.ds(c, _LANES)],
                )

            plsc.parallel_loop(0, sub * D // _LANES, unroll=8)(add_body)
            pend_out[i] = pltpu.async_copy(
                xb, o_hbm.at[pl.ds(hbm_row(i), sub)], sems_out[i % _NBUF]
            )

        for i in range(n_steps - _NBUF, n_steps):
            pend_out[i].wait()

    return k


def kernel(x, table):
    B, L, D = x.shape
    sc = _sc_add(B, L, D)
    out = sc(x.reshape(B * L, D), table)
    return out.reshape(B, L, D)
